# baseline (speedup 1.0000x reference)
import functools

import jax
import jax.numpy as jnp
from jax import lax
from jax.experimental import pallas as pl
from jax.experimental.pallas import tpu as pltpu
from jax.experimental.pallas import tpu_sc as plsc

NUM_EMB = 1_000_000
DIM = 64
B1 = 16384
B2 = 50
LANES = 128
NBLK = B1 // LANES
NUM_WORKERS = 32
BLK_PER_W = NBLK // NUM_WORKERS


def _make_sc_gather():
    mesh = plsc.VectorSubcoreMesh(core_axis_name="c", subcore_axis_name="s")

    @functools.partial(
        pl.kernel,
        out_type=jax.ShapeDtypeStruct((B2, 8, NBLK, 8, LANES), jnp.float32),
        mesh=mesh,
        scratch_types=[
            pltpu.VMEM((BLK_PER_W, B2, LANES), jnp.int32),
            pltpu.VMEM((LANES, DIM), jnp.float32),
            pltpu.VMEM((LANES, DIM), jnp.float32),
            pltpu.VMEM((DIM, LANES), jnp.float32),
            pltpu.VMEM((DIM, LANES), jnp.float32),
            pltpu.SemaphoreType.DMA,
            pltpu.SemaphoreType.DMA,
        ],
        compiler_params=pltpu.CompilerParams(use_tc_tiling_on_sc=False,
                                             needs_layout_passes=False),
    )
    def emb(idx_hbm, table_hbm, out_hbm, idx_v, rows_a, rows_b, t_a, t_b,
            sem_g, sem_o):
        wid = lax.axis_index("s") * 2 + lax.axis_index("c")
        pltpu.sync_copy(idx_hbm.at[wid], idx_v)
        rows = (rows_a, rows_b)
        tbuf = (t_a, t_b)
        lane = lax.iota(jnp.int32, 16)
        perm = [((lane + k) & 15) for k in range(16)]

        def transpose_block(src, dst):
            def rbody(i, carry):
                rv = lane + i * 16
                for j0 in range(0, DIM, 16):
                    for k in range(16):
                        jv = perm[k] + j0
                        v = plsc.load_gather(src, [rv, jv])
                        plsc.store_scatter(dst, [jv, rv], v)
                return carry
            lax.fori_loop(0, LANES // 16, rbody, 0)

        def drain(sem, shaped):
            if shaped.shape == (LANES, DIM):
                dummy = table_hbm.at[pl.ds(0, LANES)]
                pltpu.make_async_copy(dummy, shaped, sem).wait()
            else:
                for jh in range(8):
                    pltpu.make_async_copy(out_hbm.at[0, 0, 0],
                                          shaped.at[pl.ds(8 * jh, 8)],
                                          sem).wait()

        for q in range(BLK_PER_W):
            blk = wid * BLK_PER_W + q

            def fire_gather(b2, buf):
                pltpu.async_copy(table_hbm.at[idx_v.at[q, b2]], buf, sem_g)

            def fire_out(b2, buf):
                for jh in range(8):
                    pltpu.async_copy(buf.at[pl.ds(8 * jh, 8)],
                                     out_hbm.at[b2, jh, blk], sem_o)

            fire_gather(0, rows[0])

            def macro(m, carry):
                for p in range(2):
                    b2 = m * 2 + p
                    drain(sem_g, rows[p])

                    @pl.when(b2 < B2 - 1)
                    def _():
                        fire_gather(b2 + 1, rows[1 - p])

                    @pl.when(b2 >= 2)
                    def _():
                        drain(sem_o, tbuf[p])

                    transpose_block(rows[p], tbuf[p])
                    fire_out(b2, tbuf[p])
                return carry

            lax.fori_loop(0, B2 // 2, macro, 0)
            drain(sem_o, tbuf[0])
            drain(sem_o, tbuf[1])

    return emb


_sc_gather = _make_sc_gather()


def kernel(input_, weight):
    idx = (input_.astype(jnp.int32).T.reshape(B2, NBLK, LANES)
           .transpose(1, 0, 2).reshape(NUM_WORKERS, BLK_PER_W, B2, LANES))
    out5 = _sc_gather(idx, weight)
    return out5.transpose(2, 4, 0, 1, 3).reshape(B1, B2, DIM)


# speedup vs baseline: 1.4055x; 1.4055x over previous
import functools

import jax
import jax.numpy as jnp
from jax import lax
from jax.experimental import pallas as pl
from jax.experimental.pallas import tpu as pltpu
from jax.experimental.pallas import tpu_sc as plsc

NUM_EMB = 1_000_000
DIM = 64
B1 = 16384
B2 = 50
LANES = 128
NBLK = B1 // LANES
NUM_WORKERS = 32
BLK_PER_W = NBLK // NUM_WORKERS


def _make_sc_gather():
    mesh = plsc.VectorSubcoreMesh(core_axis_name="c", subcore_axis_name="s")

    @functools.partial(
        pl.kernel,
        out_type=jax.ShapeDtypeStruct((B2, 8, NBLK, 8, LANES), jnp.float32),
        mesh=mesh,
        scratch_types=[
            pltpu.VMEM((BLK_PER_W, B2, LANES), jnp.int32),
            pltpu.VMEM((LANES, DIM), jnp.float32),
            pltpu.VMEM((LANES, DIM), jnp.float32),
            pltpu.VMEM((8, 8, LANES), jnp.float32),
            pltpu.VMEM((8, 8, LANES), jnp.float32),
            pltpu.SemaphoreType.DMA,
            pltpu.SemaphoreType.DMA,
        ],
        compiler_params=pltpu.CompilerParams(use_tc_tiling_on_sc=False,
                                             needs_layout_passes=False),
    )
    def emb(idx_hbm, table_hbm, out_hbm, idx_v, rows_a, rows_b, t_a, t_b,
            sem_g, sem_o):
        wid = lax.axis_index("s") * 2 + lax.axis_index("c")
        pltpu.sync_copy(idx_hbm.at[wid], idx_v)
        rows = (rows_a, rows_b)
        tbuf = (t_a, t_b)
        lane = lax.iota(jnp.int32, 16)
        perm = [((lane + k) & 15) for k in range(16)]

        def transpose_block(src, dst):
            def rbody(i, carry):
                rv = lane + i * 16
                for j0 in range(0, DIM, 16):
                    jvs = [perm[k] + j0 for k in range(16)]
                    vs = [plsc.load_gather(src, [rv, jv]) for jv in jvs]
                    for jv, v in zip(jvs, vs):
                        plsc.store_scatter(dst, [jv >> 3, jv & 7, rv], v)
                return carry
            lax.fori_loop(0, LANES // 16, rbody, 0)

        def drain(sem, shaped):
            if shaped.shape == (LANES, DIM):
                dummy = table_hbm.at[pl.ds(0, LANES)]
            else:
                dummy = out_hbm.at[0, :, 0]
            pltpu.make_async_copy(dummy, shaped, sem).wait()

        for q in range(BLK_PER_W):
            blk = wid * BLK_PER_W + q

            def fire_gather(b2, buf):
                pltpu.async_copy(table_hbm.at[idx_v.at[q, b2]], buf, sem_g)

            def fire_out(b2, buf):
                pltpu.async_copy(buf, out_hbm.at[b2, :, blk], sem_o)

            fire_gather(0, rows[0])

            def macro(m, carry):
                for p in range(2):
                    b2 = m * 2 + p
                    drain(sem_g, rows[p])

                    @pl.when(b2 < B2 - 1)
                    def _():
                        fire_gather(b2 + 1, rows[1 - p])

                    @pl.when(b2 >= 2)
                    def _():
                        drain(sem_o, tbuf[p])

                    transpose_block(rows[p], tbuf[p])
                    fire_out(b2, tbuf[p])
                return carry

            lax.fori_loop(0, B2 // 2, macro, 0)
            drain(sem_o, tbuf[0])
            drain(sem_o, tbuf[1])

    return emb


_sc_gather = _make_sc_gather()


def kernel(input_, weight):
    idx = (input_.astype(jnp.int32).T.reshape(B2, NBLK, LANES)
           .transpose(1, 0, 2).reshape(NUM_WORKERS, BLK_PER_W, B2, LANES))
    out5 = _sc_gather(idx, weight)
    return out5.transpose(2, 4, 0, 1, 3).reshape(B1, B2, DIM)
